# Initial kernel scaffold; baseline (speedup 1.0000x reference)
import jax
import jax.numpy as jnp
from jax.experimental import pallas as pl


def kernel(ts, pos_embeddings):
    raise NotImplementedError("write your pallas kernel here")


# speedup vs baseline: 5.3802x; 5.3802x over previous
import functools

import jax
import jax.numpy as jnp
from jax import lax
from jax.experimental import pallas as pl
from jax.experimental.pallas import tpu as pltpu

_NROW = 16384
_SEQ = 200
_D = 64
_BLK = 16


def _body(ts_ref, emb_ref, out_ref):
    ts = ts_ref[...]
    b = ts.shape[0]
    tsi = lax.bitcast_convert_type(ts, jnp.int32)
    a_k = jnp.broadcast_to(tsi[:, :, None], (b, _SEQ, _SEQ))
    a_j = jnp.broadcast_to(tsi[:, None, :], (b, _SEQ, _SEQ))
    k_iota = lax.broadcasted_iota(jnp.int32, (b, _SEQ, _SEQ), 1)
    j_iota = lax.broadcasted_iota(jnp.int32, (b, _SEQ, _SEQ), 2)
    ltf = jnp.where(a_k < a_j, 1.0, 0.0)
    eqf = jnp.where(a_k == a_j, 1.0, 0.0)
    trif = jnp.where(k_iota < j_iota, 1.0, 0.0)
    cmp = ltf + eqf * trif
    rank = jnp.sum(cmp, axis=1).astype(jnp.int32)
    r_iota = lax.broadcasted_iota(jnp.int32, (b, _SEQ, _SEQ), 1)
    rr = jnp.broadcast_to(rank[:, None, :], (b, _SEQ, _SEQ))
    s = jnp.where(rr == r_iota, 1.0, 0.0).astype(jnp.bfloat16)
    emb = emb_ref[...]
    for i in range(b):
        o = jnp.dot(s[i], emb, preferred_element_type=jnp.float32)
        out_ref[i] = o[:, :_D] + o[:, _D:]


@jax.jit
def kernel(ts, pos_embeddings):
    hi = pos_embeddings.astype(jnp.bfloat16)
    lo = (pos_embeddings - hi.astype(jnp.float32)).astype(jnp.bfloat16)
    emb_cat = jnp.concatenate([hi, lo], axis=1)
    grid = _NROW // _BLK
    out = pl.pallas_call(
        _body,
        grid=(grid,),
        in_specs=[
            pl.BlockSpec((_BLK, _SEQ), lambda i: (i, 0)),
            pl.BlockSpec((_SEQ, 2 * _D), lambda i: (0, 0)),
        ],
        out_specs=pl.BlockSpec((_BLK, _SEQ, _D), lambda i: (i, 0, 0)),
        out_shape=jax.ShapeDtypeStruct((_NROW, _SEQ, _D), jnp.float32),
    )(ts, emb_cat)
    return out
